# baseline (speedup 1.0000x reference)
import functools

import jax
import jax.numpy as jnp
from jax import lax
from jax.experimental import pallas as pl
from jax.experimental.pallas import tpu as pltpu
from jax.experimental.pallas import tpu_sc as plsc

N = 100000
E = 1600000
T = 4
H = 32
HH = 16
D_IN = 128
STEPS = 8
G = 64
AUX = 2

BN = 2000
NB = N // BN

ROWS = 12800
EPAD = ROWS * 128
RPT = ROWS // 16
CH = 8
CHUNKS = RPT // CH
NPAD = 100096
TROWS = NPAD // 16

_f32 = jnp.float32


def _sc_edge_body(acts, gidx, dstx, zer, out,
                  gidx_v, dst_v, rows_v, accum, gsem, ssem):
    sid = lax.axis_index("s")
    pltpu.sync_copy(zer.at[pl.ds(sid * TROWS, TROWS)],
                    accum.at[pl.ds(sid * TROWS, TROWS)])
    plsc.subcore_barrier()

    def chunk(gi, c):
        row0 = sid * RPT + gi * CH
        pltpu.sync_copy(gidx.at[pl.ds(row0, CH)], gidx_v)
        pltpu.sync_copy(dstx.at[pl.ds(row0, CH)], dst_v)
        gds = [pltpu.async_copy(acts.at[gidx_v.at[b]], rows_v.at[b], gsem)
               for b in range(CH)]
        for d in gds:
            d.wait()
        sds = [pltpu.async_copy(rows_v.at[b], accum.at[dst_v.at[b]],
                                ssem, add=True)
               for b in range(CH)]
        for d in sds:
            d.wait()
        return c

    lax.fori_loop(0, CHUNKS, chunk, 0)
    plsc.subcore_barrier()
    pltpu.sync_copy(accum.at[pl.ds(sid * TROWS, TROWS)],
                    out.at[pl.ds(sid * TROWS, TROWS)])


@functools.cache
def _get_sc_edge():
    mesh = plsc.VectorSubcoreMesh(
        core_axis_name="c", subcore_axis_name="s",
        num_cores=1, num_subcores=16)
    return pl.kernel(
        _sc_edge_body,
        out_type=jax.ShapeDtypeStruct((NPAD, HH), _f32),
        mesh=mesh,
        scratch_types=[
            pltpu.VMEM((CH, 128), jnp.int32),
            pltpu.VMEM((CH, 128), jnp.int32),
            pltpu.VMEM((CH, 128, HH), _f32),
            pltpu.VMEM_SHARED((NPAD, HH), _f32),
            pltpu.SemaphoreType.DMA,
            pltpu.SemaphoreType.DMA,
        ],
        compiler_params=pltpu.CompilerParams(use_tc_tiling_on_sc=False),
    )


def _acts_out(h, ew, alo_ref, ahi_ref):
    for t in range(T):
        a = jnp.dot(h, ew[t], preferred_element_type=_f32)
        alo_ref[t] = a[:, :HH]
        ahi_ref[t] = a[:, HH:]


def _embed_body(x_ref, w0, b0, w1, b1, ew_ref, h_ref, alo_ref, ahi_ref):
    h = jax.nn.relu(jnp.dot(x_ref[...], w0[...],
                            preferred_element_type=_f32) + b0[...])
    h = jax.nn.relu(jnp.dot(h, w1[...],
                            preferred_element_type=_f32) + b1[...])
    h_ref[...] = h
    _acts_out(h, ew_ref[...], alo_ref, ahi_ref)


def _full(shape):
    nd = len(shape)
    return pl.BlockSpec(shape, lambda i, _nd=nd: (0,) * _nd)


_embed_call = pl.pallas_call(
    _embed_body,
    grid=(NB,),
    in_specs=[
        pl.BlockSpec((BN, D_IN), lambda i: (i, 0)),
        _full((D_IN, H)), _full((1, H)), _full((H, H)), _full((1, H)),
        _full((T, H, H)),
    ],
    out_specs=[
        pl.BlockSpec((BN, H), lambda i: (i, 0)),
        pl.BlockSpec((T, BN, HH), lambda i: (0, i, 0)),
        pl.BlockSpec((T, BN, HH), lambda i: (0, i, 0)),
    ],
    out_shape=[
        jax.ShapeDtypeStruct((N, H), _f32),
        jax.ShapeDtypeStruct((T, N, HH), _f32),
        jax.ShapeDtypeStruct((T, N, HH), _f32),
    ],
)


def _gru_math(h_ref, alo_ref, ahi_ref, wz, uz, bz, wr, ur, br, wh, uh, bh):
    h = h_ref[...]
    al = alo_ref[...]
    ah = ahi_ref[...]

    def am(w_ref):
        w = w_ref[...]
        return (jnp.dot(al, w[:HH], preferred_element_type=_f32)
                + jnp.dot(ah, w[HH:], preferred_element_type=_f32))

    def hm(v, w_ref):
        return jnp.dot(v, w_ref[...], preferred_element_type=_f32)

    z = jax.nn.sigmoid(am(wz) + hm(h, uz) + bz[...])
    r = jax.nn.sigmoid(am(wr) + hm(h, ur) + br[...])
    hh = jnp.tanh(am(wh) + hm(r * h, uh) + bh[...])
    return (1.0 - z) * h + z * hh


def _gru_acts_body(h_ref, alo_ref, ahi_ref, wz, uz, bz, wr, ur, br,
                   wh, uh, bh, ew_ref, ho_ref, aol_ref, aoh_ref):
    hn = _gru_math(h_ref, alo_ref, ahi_ref, wz, uz, bz, wr, ur, br, wh, uh, bh)
    ho_ref[...] = hn
    _acts_out(hn, ew_ref[...], aol_ref, aoh_ref)


def _gru_last_body(h_ref, alo_ref, ahi_ref, wz, uz, bz, wr, ur, br,
                   wh, uh, bh, ho_ref):
    ho_ref[...] = _gru_math(h_ref, alo_ref, ahi_ref,
                            wz, uz, bz, wr, ur, br, wh, uh, bh)


_gru_in_specs = [
    pl.BlockSpec((BN, H), lambda i: (i, 0)),
    pl.BlockSpec((BN, HH), lambda i: (i, 0)),
    pl.BlockSpec((BN, HH), lambda i: (i, 0)),
] + [_full((H, H)), _full((H, H)), _full((1, H))] * 3

_gru_acts_call = pl.pallas_call(
    _gru_acts_body,
    grid=(NB,),
    in_specs=_gru_in_specs + [_full((T, H, H))],
    out_specs=[
        pl.BlockSpec((BN, H), lambda i: (i, 0)),
        pl.BlockSpec((T, BN, HH), lambda i: (0, i, 0)),
        pl.BlockSpec((T, BN, HH), lambda i: (0, i, 0)),
    ],
    out_shape=[
        jax.ShapeDtypeStruct((N, H), _f32),
        jax.ShapeDtypeStruct((T, N, HH), _f32),
        jax.ShapeDtypeStruct((T, N, HH), _f32),
    ],
)

_gru_last_call = pl.pallas_call(
    _gru_last_body,
    grid=(NB,),
    in_specs=list(_gru_in_specs),
    out_specs=pl.BlockSpec((BN, H), lambda i: (i, 0)),
    out_shape=jax.ShapeDtypeStruct((N, H), _f32),
)


def _mlp3(v, w0, b0, w1, b1, w2, b2):
    v = jax.nn.relu(jnp.dot(v, w0[...], preferred_element_type=_f32) + b0[...])
    v = jax.nn.relu(jnp.dot(v, w1[...], preferred_element_type=_f32) + b1[...])
    return jnp.dot(v, w2[...], preferred_element_type=_f32) + b2[...]


def _readout_body(h_ref, gid_ref, fw0, fb0, fw1, fb1, fw2, fb2,
                  gw0, gb0, gw1, gb1, gw2, gb2, ge_ref):
    h = h_ref[...]
    f = _mlp3(h, fw0, fb0, fw1, fb1, fw2, fb2)
    g = jax.nn.sigmoid(_mlp3(h, gw0, gb0, gw1, gb1, gw2, gb2))
    gated = g * f
    ids = gid_ref[...]
    onehot = (ids == lax.broadcasted_iota(jnp.int32, (BN, G), 1)).astype(_f32)
    part = lax.dot_general(onehot, gated, (((0,), (0,)), ((), ())),
                           preferred_element_type=_f32)

    @pl.when(pl.program_id(0) == 0)
    def _():
        ge_ref[...] = jnp.zeros_like(ge_ref)

    ge_ref[...] += part


_readout_call = pl.pallas_call(
    _readout_body,
    grid=(NB,),
    in_specs=[
        pl.BlockSpec((BN, H), lambda i: (i, 0)),
        pl.BlockSpec((BN, 1), lambda i: (i, 0)),
        _full((H, H)), _full((1, H)), _full((H, H)), _full((1, H)),
        _full((H, G)), _full((1, G)),
        _full((H, H)), _full((1, H)), _full((H, H)), _full((1, H)),
        _full((H, G)), _full((1, G)),
    ],
    out_specs=pl.BlockSpec((G, G), lambda i: (0, 0)),
    out_shape=jax.ShapeDtypeStruct((G, G), _f32),
)


def _final_body(ge_ref, aux_ref, rw0, rb0, rw1, rb1, rw2, rb2,
                a1wa, a1wb, a1b, a2w, a2b, out_ref):
    r1 = _mlp3(ge_ref[...], rw0, rb0, rw1, rb1, rw2, rb2)
    a1 = jax.nn.relu(jnp.dot(r1, a1wa[...], preferred_element_type=_f32)
                     + jnp.dot(aux_ref[...], a1wb[...],
                               preferred_element_type=_f32)
                     + a1b[...])
    out_ref[...] = jax.nn.sigmoid(
        jnp.dot(a1, a2w[...], preferred_element_type=_f32) + a2b[...])


_final_call = pl.pallas_call(
    _final_body,
    out_shape=jax.ShapeDtypeStruct((G, AUX), _f32),
)


def kernel(x, edge_index, edge_type, graph_ids, aux_in, params):
    p = params

    def b2(v):
        return v.reshape(1, -1)

    src = edge_index[0]
    dst = edge_index[1]
    gidx = edge_type * N + src
    padn = EPAD - E
    gidx2 = jnp.concatenate(
        [gidx, jnp.zeros((padn,), jnp.int32)]).reshape(ROWS, 128)
    dst2 = jnp.concatenate(
        [dst, jnp.full((padn,), N, jnp.int32)]).reshape(ROWS, 128)
    zer = jnp.zeros((NPAD, HH), _f32)

    h, alo3, ahi3 = _embed_call(x, p['emb_W0'], b2(p['emb_b0']),
                                p['emb_W1'], b2(p['emb_b1']), p['edge_W'])
    gw = (p['Wz'], p['Uz'], b2(p['bz']), p['Wr'], p['Ur'], b2(p['br']),
          p['Wh'], p['Uh'], b2(p['bh']))
    sc_edge = _get_sc_edge()
    for s in range(STEPS):
        agg_lo = sc_edge(alo3.reshape(T * N, HH), gidx2, dst2, zer)
        agg_hi = sc_edge(ahi3.reshape(T * N, HH), gidx2, dst2, zer)
        agg_lo = agg_lo[:N]
        agg_hi = agg_hi[:N]
        if s < STEPS - 1:
            h, alo3, ahi3 = _gru_acts_call(h, agg_lo, agg_hi, *gw,
                                           p['edge_W'])
        else:
            h = _gru_last_call(h, agg_lo, agg_hi, *gw)

    ge = _readout_call(h, graph_ids.reshape(N, 1),
                       p['fm_W0'], b2(p['fm_b0']), p['fm_W1'], b2(p['fm_b1']),
                       p['fm_W2'], b2(p['fm_b2']),
                       p['gm_W0'], b2(p['gm_b0']), p['gm_W1'], b2(p['gm_b1']),
                       p['gm_W2'], b2(p['gm_b2']))
    return _final_call(ge, aux_in,
                       p['red_W0'], b2(p['red_b0']), p['red_W1'],
                       b2(p['red_b1']), p['red_W2'], b2(p['red_b2']),
                       p['aux1_W'][:G], p['aux1_W'][G:], b2(p['aux1_b']),
                       p['aux2_W'], b2(p['aux2_b']))


# speedup vs baseline: 1.4418x; 1.4418x over previous
import functools

import jax
import jax.numpy as jnp
from jax import lax
from jax.experimental import pallas as pl
from jax.experimental.pallas import tpu as pltpu
from jax.experimental.pallas import tpu_sc as plsc

N = 100000
E = 1600000
T = 4
H = 32
HH = 16
D_IN = 128
STEPS = 8
G = 64
AUX = 2

BN = 2000
NB = N // BN

ROWS = 12800
EPAD = ROWS * 128
RPT = ROWS // 16
CH = 5
CHUNKS = RPT // CH
TPAIRS = CHUNKS // 2
NPAD = 100096
TROWS = NPAD // 16

_f32 = jnp.float32


def _sc_edge_body(alo, ahi, ia, zer, olo, ohi,
                  ibufA, ibufB, rbufA, rbufB, accum, isem, gsem, ssem):
    cid = lax.axis_index("c")

    def half(acts, out):
        sid = lax.axis_index("s")
        pltpu.sync_copy(zer.at[pl.ds(sid * TROWS, TROWS)],
                        accum.at[pl.ds(sid * TROWS, TROWS)])
        plsc.subcore_barrier()
        base = sid * RPT

        def i_fire(ib, c):
            return pltpu.async_copy(ia.at[pl.ds(base + c * CH, CH)], ib, isem)

        def i_wait(ib, c):
            pltpu.make_async_copy(
                ia.at[pl.ds(base + c * CH, CH)], ib, isem).wait()

        def g_fire(ib, rb):
            for b in range(CH):
                pltpu.async_copy(acts.at[ib.at[b, 0]], rb.at[b], gsem)

        def g_wait(ib, rb):
            for b in range(CH):
                pltpu.make_async_copy(
                    acts.at[ib.at[b, 0]], rb.at[b], gsem).wait()

        def s_fire(ib, rb):
            for b in range(CH):
                pltpu.async_copy(rb.at[b], accum.at[ib.at[b, 1]],
                                 ssem, add=True)

        def s_wait(ib, rb):
            for b in range(CH):
                pltpu.make_async_copy(
                    rb.at[b], accum.at[ib.at[b, 1]], ssem).wait()

        i_fire(ibufA, 0).wait()
        g_fire(ibufA, rbufA)

        def body(t, c):
            b = 2 * t + 1

            @pl.when(t > 0)
            def _():
                s_wait(ibufB, rbufB)

            i_fire(ibufB, b)
            g_wait(ibufA, rbufA)
            s_fire(ibufA, rbufA)
            i_wait(ibufB, b)
            g_fire(ibufB, rbufB)
            s_wait(ibufA, rbufA)

            @pl.when(t < TPAIRS - 1)
            def _():
                i_fire(ibufA, b + 1)

            g_wait(ibufB, rbufB)
            s_fire(ibufB, rbufB)

            @pl.when(t < TPAIRS - 1)
            def _():
                i_wait(ibufA, b + 1)
                g_fire(ibufA, rbufA)
            return c

        lax.fori_loop(0, TPAIRS, body, 0)
        s_wait(ibufB, rbufB)
        plsc.subcore_barrier()
        pltpu.sync_copy(accum.at[pl.ds(sid * TROWS, TROWS)],
                        out.at[pl.ds(sid * TROWS, TROWS)])

    @pl.when(cid == 0)
    def _():
        half(alo, olo)

    @pl.when(cid == 1)
    def _():
        half(ahi, ohi)


@functools.cache
def _get_sc_edge():
    mesh = plsc.VectorSubcoreMesh(
        core_axis_name="c", subcore_axis_name="s",
        num_cores=2, num_subcores=16)
    return pl.kernel(
        _sc_edge_body,
        out_type=[jax.ShapeDtypeStruct((NPAD, HH), _f32),
                  jax.ShapeDtypeStruct((NPAD, HH), _f32)],
        mesh=mesh,
        scratch_types=[
            pltpu.VMEM((CH, 2, 128), jnp.int32),
            pltpu.VMEM((CH, 2, 128), jnp.int32),
            pltpu.VMEM((CH, 128, HH), _f32),
            pltpu.VMEM((CH, 128, HH), _f32),
            pltpu.VMEM_SHARED((NPAD, HH), _f32),
            pltpu.SemaphoreType.DMA,
            pltpu.SemaphoreType.DMA,
            pltpu.SemaphoreType.DMA,
        ],
        compiler_params=pltpu.CompilerParams(use_tc_tiling_on_sc=False),
    )


def _acts_out(h, ew, alo_ref, ahi_ref):
    for t in range(T):
        a = jnp.dot(h, ew[t], preferred_element_type=_f32)
        alo_ref[t] = a[:, :HH]
        ahi_ref[t] = a[:, HH:]


def _embed_body(x_ref, w0, b0, w1, b1, ew_ref, h_ref, alo_ref, ahi_ref):
    h = jax.nn.relu(jnp.dot(x_ref[...], w0[...],
                            preferred_element_type=_f32) + b0[...])
    h = jax.nn.relu(jnp.dot(h, w1[...],
                            preferred_element_type=_f32) + b1[...])
    h_ref[...] = h
    _acts_out(h, ew_ref[...], alo_ref, ahi_ref)


def _full(shape):
    nd = len(shape)
    return pl.BlockSpec(shape, lambda i, _nd=nd: (0,) * _nd)


_embed_call = pl.pallas_call(
    _embed_body,
    grid=(NB,),
    in_specs=[
        pl.BlockSpec((BN, D_IN), lambda i: (i, 0)),
        _full((D_IN, H)), _full((1, H)), _full((H, H)), _full((1, H)),
        _full((T, H, H)),
    ],
    out_specs=[
        pl.BlockSpec((BN, H), lambda i: (i, 0)),
        pl.BlockSpec((T, BN, HH), lambda i: (0, i, 0)),
        pl.BlockSpec((T, BN, HH), lambda i: (0, i, 0)),
    ],
    out_shape=[
        jax.ShapeDtypeStruct((N, H), _f32),
        jax.ShapeDtypeStruct((T, N, HH), _f32),
        jax.ShapeDtypeStruct((T, N, HH), _f32),
    ],
)


def _gru_math(h_ref, alo_ref, ahi_ref, wz, uz, bz, wr, ur, br, wh, uh, bh):
    h = h_ref[...]
    al = alo_ref[...]
    ah = ahi_ref[...]

    def am(w_ref):
        w = w_ref[...]
        return (jnp.dot(al, w[:HH], preferred_element_type=_f32)
                + jnp.dot(ah, w[HH:], preferred_element_type=_f32))

    def hm(v, w_ref):
        return jnp.dot(v, w_ref[...], preferred_element_type=_f32)

    z = jax.nn.sigmoid(am(wz) + hm(h, uz) + bz[...])
    r = jax.nn.sigmoid(am(wr) + hm(h, ur) + br[...])
    hh = jnp.tanh(am(wh) + hm(r * h, uh) + bh[...])
    return (1.0 - z) * h + z * hh


def _gru_acts_body(h_ref, alo_ref, ahi_ref, wz, uz, bz, wr, ur, br,
                   wh, uh, bh, ew_ref, ho_ref, aol_ref, aoh_ref):
    hn = _gru_math(h_ref, alo_ref, ahi_ref, wz, uz, bz, wr, ur, br, wh, uh, bh)
    ho_ref[...] = hn
    _acts_out(hn, ew_ref[...], aol_ref, aoh_ref)


def _gru_last_body(h_ref, alo_ref, ahi_ref, wz, uz, bz, wr, ur, br,
                   wh, uh, bh, ho_ref):
    ho_ref[...] = _gru_math(h_ref, alo_ref, ahi_ref,
                            wz, uz, bz, wr, ur, br, wh, uh, bh)


_gru_in_specs = [
    pl.BlockSpec((BN, H), lambda i: (i, 0)),
    pl.BlockSpec((BN, HH), lambda i: (i, 0)),
    pl.BlockSpec((BN, HH), lambda i: (i, 0)),
] + [_full((H, H)), _full((H, H)), _full((1, H))] * 3

_gru_acts_call = pl.pallas_call(
    _gru_acts_body,
    grid=(NB,),
    in_specs=_gru_in_specs + [_full((T, H, H))],
    out_specs=[
        pl.BlockSpec((BN, H), lambda i: (i, 0)),
        pl.BlockSpec((T, BN, HH), lambda i: (0, i, 0)),
        pl.BlockSpec((T, BN, HH), lambda i: (0, i, 0)),
    ],
    out_shape=[
        jax.ShapeDtypeStruct((N, H), _f32),
        jax.ShapeDtypeStruct((T, N, HH), _f32),
        jax.ShapeDtypeStruct((T, N, HH), _f32),
    ],
)

_gru_last_call = pl.pallas_call(
    _gru_last_body,
    grid=(NB,),
    in_specs=list(_gru_in_specs),
    out_specs=pl.BlockSpec((BN, H), lambda i: (i, 0)),
    out_shape=jax.ShapeDtypeStruct((N, H), _f32),
)


def _mlp3(v, w0, b0, w1, b1, w2, b2):
    v = jax.nn.relu(jnp.dot(v, w0[...], preferred_element_type=_f32) + b0[...])
    v = jax.nn.relu(jnp.dot(v, w1[...], preferred_element_type=_f32) + b1[...])
    return jnp.dot(v, w2[...], preferred_element_type=_f32) + b2[...]


def _readout_body(h_ref, gid_ref, fw0, fb0, fw1, fb1, fw2, fb2,
                  gw0, gb0, gw1, gb1, gw2, gb2, ge_ref):
    h = h_ref[...]
    f = _mlp3(h, fw0, fb0, fw1, fb1, fw2, fb2)
    g = jax.nn.sigmoid(_mlp3(h, gw0, gb0, gw1, gb1, gw2, gb2))
    gated = g * f
    ids = gid_ref[...]
    onehot = (ids == lax.broadcasted_iota(jnp.int32, (BN, G), 1)).astype(_f32)
    part = lax.dot_general(onehot, gated, (((0,), (0,)), ((), ())),
                           preferred_element_type=_f32)

    @pl.when(pl.program_id(0) == 0)
    def _():
        ge_ref[...] = jnp.zeros_like(ge_ref)

    ge_ref[...] += part


_readout_call = pl.pallas_call(
    _readout_body,
    grid=(NB,),
    in_specs=[
        pl.BlockSpec((BN, H), lambda i: (i, 0)),
        pl.BlockSpec((BN, 1), lambda i: (i, 0)),
        _full((H, H)), _full((1, H)), _full((H, H)), _full((1, H)),
        _full((H, G)), _full((1, G)),
        _full((H, H)), _full((1, H)), _full((H, H)), _full((1, H)),
        _full((H, G)), _full((1, G)),
    ],
    out_specs=pl.BlockSpec((G, G), lambda i: (0, 0)),
    out_shape=jax.ShapeDtypeStruct((G, G), _f32),
)


def _final_body(ge_ref, aux_ref, rw0, rb0, rw1, rb1, rw2, rb2,
                a1wa, a1wb, a1b, a2w, a2b, out_ref):
    r1 = _mlp3(ge_ref[...], rw0, rb0, rw1, rb1, rw2, rb2)
    a1 = jax.nn.relu(jnp.dot(r1, a1wa[...], preferred_element_type=_f32)
                     + jnp.dot(aux_ref[...], a1wb[...],
                               preferred_element_type=_f32)
                     + a1b[...])
    out_ref[...] = jax.nn.sigmoid(
        jnp.dot(a1, a2w[...], preferred_element_type=_f32) + a2b[...])


_final_call = pl.pallas_call(
    _final_body,
    out_shape=jax.ShapeDtypeStruct((G, AUX), _f32),
)


def kernel(x, edge_index, edge_type, graph_ids, aux_in, params):
    p = params

    def b2(v):
        return v.reshape(1, -1)

    src = edge_index[0]
    dst = edge_index[1]
    gidx = edge_type * N + src
    padn = EPAD - E
    gidx2 = jnp.concatenate(
        [gidx, jnp.zeros((padn,), jnp.int32)]).reshape(ROWS, 128)
    dst2 = jnp.concatenate(
        [dst, jnp.full((padn,), N, jnp.int32)]).reshape(ROWS, 128)
    ia = jnp.stack([gidx2, dst2], axis=1)
    zer = jnp.zeros((NPAD, HH), _f32)

    h, alo3, ahi3 = _embed_call(x, p['emb_W0'], b2(p['emb_b0']),
                                p['emb_W1'], b2(p['emb_b1']), p['edge_W'])
    gw = (p['Wz'], p['Uz'], b2(p['bz']), p['Wr'], p['Ur'], b2(p['br']),
          p['Wh'], p['Uh'], b2(p['bh']))
    sc_edge = _get_sc_edge()
    for s in range(STEPS):
        agg_lo, agg_hi = sc_edge(alo3.reshape(T * N, HH),
                                 ahi3.reshape(T * N, HH), ia, zer)
        agg_lo = agg_lo[:N]
        agg_hi = agg_hi[:N]
        if s < STEPS - 1:
            h, alo3, ahi3 = _gru_acts_call(h, agg_lo, agg_hi, *gw,
                                           p['edge_W'])
        else:
            h = _gru_last_call(h, agg_lo, agg_hi, *gw)

    ge = _readout_call(h, graph_ids.reshape(N, 1),
                       p['fm_W0'], b2(p['fm_b0']), p['fm_W1'], b2(p['fm_b1']),
                       p['fm_W2'], b2(p['fm_b2']),
                       p['gm_W0'], b2(p['gm_b0']), p['gm_W1'], b2(p['gm_b1']),
                       p['gm_W2'], b2(p['gm_b2']))
    return _final_call(ge, aux_in,
                       p['red_W0'], b2(p['red_b0']), p['red_W1'],
                       b2(p['red_b1']), p['red_W2'], b2(p['red_b2']),
                       p['aux1_W'][:G], p['aux1_W'][G:], b2(p['aux1_b']),
                       p['aux2_W'], b2(p['aux2_b']))


# speedup vs baseline: 1.4742x; 1.0225x over previous
import functools

import jax
import jax.numpy as jnp
from jax import lax
from jax.experimental import pallas as pl
from jax.experimental.pallas import tpu as pltpu
from jax.experimental.pallas import tpu_sc as plsc

N = 100000
E = 1600000
T = 4
H = 32
HH = 16
D_IN = 128
STEPS = 8
G = 64
AUX = 2

BN = 2000
NB = N // BN

ROWS = 12800
EPAD = ROWS * 128
RPT = ROWS // 16
CH = 5
CHUNKS = RPT // CH
TPAIRS = CHUNKS // 2
NPAD = 100096
TROWS = NPAD // 16

_f32 = jnp.float32


def _sc_edge_body(alo, ahi, ia, zer, olo, ohi,
                  gbufA, dbufA, gbufB, dbufB, rbufA, rbufB,
                  accum, isem, gsem, ssem):
    cid = lax.axis_index("c")

    def half(acts, out):
        sid = lax.axis_index("s")
        pltpu.sync_copy(zer.at[pl.ds(sid * TROWS, TROWS)],
                        accum.at[pl.ds(sid * TROWS, TROWS)])
        plsc.subcore_barrier()
        base = sid * RPT

        CE = CH * 128

        def i_fire(gb, db, c):
            pltpu.async_copy(ia.at[0, pl.ds((base + c * CH) * 128, CE)],
                             gb, isem)
            pltpu.async_copy(ia.at[1, pl.ds((base + c * CH) * 128, CE)],
                             db, isem)

        def i_wait(gb, db, c):
            pltpu.make_async_copy(
                ia.at[0, pl.ds((base + c * CH) * 128, CE)], gb, isem).wait()
            pltpu.make_async_copy(
                ia.at[1, pl.ds((base + c * CH) * 128, CE)], db, isem).wait()

        def g_fire(gb, rb):
            pltpu.async_copy(acts.at[gb], rb, gsem)

        def g_wait(gb, rb):
            pltpu.make_async_copy(acts.at[gb], rb, gsem).wait()

        def s_fire(db, rb):
            pltpu.async_copy(rb, accum.at[db], ssem, add=True)

        def s_wait(db, rb):
            pltpu.make_async_copy(rb, accum.at[db], ssem).wait()

        i_fire(gbufA, dbufA, 0)
        i_wait(gbufA, dbufA, 0)
        g_fire(gbufA, rbufA)

        def body(t, c):
            b = 2 * t + 1

            @pl.when(t > 0)
            def _():
                s_wait(dbufB, rbufB)

            i_fire(gbufB, dbufB, b)
            g_wait(gbufA, rbufA)
            s_fire(dbufA, rbufA)
            i_wait(gbufB, dbufB, b)
            g_fire(gbufB, rbufB)
            s_wait(dbufA, rbufA)

            @pl.when(t < TPAIRS - 1)
            def _():
                i_fire(gbufA, dbufA, b + 1)

            g_wait(gbufB, rbufB)
            s_fire(dbufB, rbufB)

            @pl.when(t < TPAIRS - 1)
            def _():
                i_wait(gbufA, dbufA, b + 1)
                g_fire(gbufA, rbufA)
            return c

        lax.fori_loop(0, TPAIRS, body, 0)
        s_wait(dbufB, rbufB)
        plsc.subcore_barrier()
        pltpu.sync_copy(accum.at[pl.ds(sid * TROWS, TROWS)],
                        out.at[pl.ds(sid * TROWS, TROWS)])

    @pl.when(cid == 0)
    def _():
        half(alo, olo)

    @pl.when(cid == 1)
    def _():
        half(ahi, ohi)


@functools.cache
def _get_sc_edge():
    mesh = plsc.VectorSubcoreMesh(
        core_axis_name="c", subcore_axis_name="s",
        num_cores=2, num_subcores=16)
    return pl.kernel(
        _sc_edge_body,
        out_type=[jax.ShapeDtypeStruct((NPAD, HH), _f32),
                  jax.ShapeDtypeStruct((NPAD, HH), _f32)],
        mesh=mesh,
        scratch_types=[
            pltpu.VMEM((CH * 128,), jnp.int32),
            pltpu.VMEM((CH * 128,), jnp.int32),
            pltpu.VMEM((CH * 128,), jnp.int32),
            pltpu.VMEM((CH * 128,), jnp.int32),
            pltpu.VMEM((CH * 128, HH), _f32),
            pltpu.VMEM((CH * 128, HH), _f32),
            pltpu.VMEM_SHARED((NPAD, HH), _f32),
            pltpu.SemaphoreType.DMA,
            pltpu.SemaphoreType.DMA,
            pltpu.SemaphoreType.DMA,
        ],
        compiler_params=pltpu.CompilerParams(use_tc_tiling_on_sc=False),
    )


def _acts_out(h, ew, alo_ref, ahi_ref):
    for t in range(T):
        a = jnp.dot(h, ew[t], preferred_element_type=_f32)
        alo_ref[t] = a[:, :HH]
        ahi_ref[t] = a[:, HH:]


def _embed_body(x_ref, w0, b0, w1, b1, ew_ref, h_ref, alo_ref, ahi_ref):
    h = jax.nn.relu(jnp.dot(x_ref[...], w0[...],
                            preferred_element_type=_f32) + b0[...])
    h = jax.nn.relu(jnp.dot(h, w1[...],
                            preferred_element_type=_f32) + b1[...])
    h_ref[...] = h
    _acts_out(h, ew_ref[...], alo_ref, ahi_ref)


def _full(shape):
    nd = len(shape)
    return pl.BlockSpec(shape, lambda i, _nd=nd: (0,) * _nd)


_embed_call = pl.pallas_call(
    _embed_body,
    grid=(NB,),
    in_specs=[
        pl.BlockSpec((BN, D_IN), lambda i: (i, 0)),
        _full((D_IN, H)), _full((1, H)), _full((H, H)), _full((1, H)),
        _full((T, H, H)),
    ],
    out_specs=[
        pl.BlockSpec((BN, H), lambda i: (i, 0)),
        pl.BlockSpec((T, BN, HH), lambda i: (0, i, 0)),
        pl.BlockSpec((T, BN, HH), lambda i: (0, i, 0)),
    ],
    out_shape=[
        jax.ShapeDtypeStruct((N, H), _f32),
        jax.ShapeDtypeStruct((T, N, HH), _f32),
        jax.ShapeDtypeStruct((T, N, HH), _f32),
    ],
)


def _gru_math(h_ref, alo_ref, ahi_ref, wz, uz, bz, wr, ur, br, wh, uh, bh):
    h = h_ref[...]
    al = alo_ref[...]
    ah = ahi_ref[...]

    def am(w_ref):
        w = w_ref[...]
        return (jnp.dot(al, w[:HH], preferred_element_type=_f32)
                + jnp.dot(ah, w[HH:], preferred_element_type=_f32))

    def hm(v, w_ref):
        return jnp.dot(v, w_ref[...], preferred_element_type=_f32)

    z = jax.nn.sigmoid(am(wz) + hm(h, uz) + bz[...])
    r = jax.nn.sigmoid(am(wr) + hm(h, ur) + br[...])
    hh = jnp.tanh(am(wh) + hm(r * h, uh) + bh[...])
    return (1.0 - z) * h + z * hh


def _gru_acts_body(h_ref, alo_ref, ahi_ref, wz, uz, bz, wr, ur, br,
                   wh, uh, bh, ew_ref, ho_ref, aol_ref, aoh_ref):
    hn = _gru_math(h_ref, alo_ref, ahi_ref, wz, uz, bz, wr, ur, br, wh, uh, bh)
    ho_ref[...] = hn
    _acts_out(hn, ew_ref[...], aol_ref, aoh_ref)


def _gru_last_body(h_ref, alo_ref, ahi_ref, wz, uz, bz, wr, ur, br,
                   wh, uh, bh, ho_ref):
    ho_ref[...] = _gru_math(h_ref, alo_ref, ahi_ref,
                            wz, uz, bz, wr, ur, br, wh, uh, bh)


_gru_in_specs = [
    pl.BlockSpec((BN, H), lambda i: (i, 0)),
    pl.BlockSpec((BN, HH), lambda i: (i, 0)),
    pl.BlockSpec((BN, HH), lambda i: (i, 0)),
] + [_full((H, H)), _full((H, H)), _full((1, H))] * 3

_gru_acts_call = pl.pallas_call(
    _gru_acts_body,
    grid=(NB,),
    in_specs=_gru_in_specs + [_full((T, H, H))],
    out_specs=[
        pl.BlockSpec((BN, H), lambda i: (i, 0)),
        pl.BlockSpec((T, BN, HH), lambda i: (0, i, 0)),
        pl.BlockSpec((T, BN, HH), lambda i: (0, i, 0)),
    ],
    out_shape=[
        jax.ShapeDtypeStruct((N, H), _f32),
        jax.ShapeDtypeStruct((T, N, HH), _f32),
        jax.ShapeDtypeStruct((T, N, HH), _f32),
    ],
)

_gru_last_call = pl.pallas_call(
    _gru_last_body,
    grid=(NB,),
    in_specs=list(_gru_in_specs),
    out_specs=pl.BlockSpec((BN, H), lambda i: (i, 0)),
    out_shape=jax.ShapeDtypeStruct((N, H), _f32),
)


def _mlp3(v, w0, b0, w1, b1, w2, b2):
    v = jax.nn.relu(jnp.dot(v, w0[...], preferred_element_type=_f32) + b0[...])
    v = jax.nn.relu(jnp.dot(v, w1[...], preferred_element_type=_f32) + b1[...])
    return jnp.dot(v, w2[...], preferred_element_type=_f32) + b2[...]


def _readout_body(h_ref, gid_ref, fw0, fb0, fw1, fb1, fw2, fb2,
                  gw0, gb0, gw1, gb1, gw2, gb2, ge_ref):
    h = h_ref[...]
    f = _mlp3(h, fw0, fb0, fw1, fb1, fw2, fb2)
    g = jax.nn.sigmoid(_mlp3(h, gw0, gb0, gw1, gb1, gw2, gb2))
    gated = g * f
    ids = gid_ref[...]
    onehot = (ids == lax.broadcasted_iota(jnp.int32, (BN, G), 1)).astype(_f32)
    part = lax.dot_general(onehot, gated, (((0,), (0,)), ((), ())),
                           preferred_element_type=_f32)

    @pl.when(pl.program_id(0) == 0)
    def _():
        ge_ref[...] = jnp.zeros_like(ge_ref)

    ge_ref[...] += part


_readout_call = pl.pallas_call(
    _readout_body,
    grid=(NB,),
    in_specs=[
        pl.BlockSpec((BN, H), lambda i: (i, 0)),
        pl.BlockSpec((BN, 1), lambda i: (i, 0)),
        _full((H, H)), _full((1, H)), _full((H, H)), _full((1, H)),
        _full((H, G)), _full((1, G)),
        _full((H, H)), _full((1, H)), _full((H, H)), _full((1, H)),
        _full((H, G)), _full((1, G)),
    ],
    out_specs=pl.BlockSpec((G, G), lambda i: (0, 0)),
    out_shape=jax.ShapeDtypeStruct((G, G), _f32),
)


def _final_body(ge_ref, aux_ref, rw0, rb0, rw1, rb1, rw2, rb2,
                a1wa, a1wb, a1b, a2w, a2b, out_ref):
    r1 = _mlp3(ge_ref[...], rw0, rb0, rw1, rb1, rw2, rb2)
    a1 = jax.nn.relu(jnp.dot(r1, a1wa[...], preferred_element_type=_f32)
                     + jnp.dot(aux_ref[...], a1wb[...],
                               preferred_element_type=_f32)
                     + a1b[...])
    out_ref[...] = jax.nn.sigmoid(
        jnp.dot(a1, a2w[...], preferred_element_type=_f32) + a2b[...])


_final_call = pl.pallas_call(
    _final_body,
    out_shape=jax.ShapeDtypeStruct((G, AUX), _f32),
)


def kernel(x, edge_index, edge_type, graph_ids, aux_in, params):
    p = params

    def b2(v):
        return v.reshape(1, -1)

    src = edge_index[0]
    dst = edge_index[1]
    gidx = edge_type * N + src
    padn = EPAD - E
    gidx2 = jnp.concatenate(
        [gidx, jnp.zeros((padn,), jnp.int32)]).reshape(ROWS, 128)
    dst2 = jnp.concatenate(
        [dst, jnp.full((padn,), N, jnp.int32)]).reshape(ROWS, 128)
    ia = jnp.stack([gidx2.reshape(-1), dst2.reshape(-1)], axis=0)
    zer = jnp.zeros((NPAD, HH), _f32)

    h, alo3, ahi3 = _embed_call(x, p['emb_W0'], b2(p['emb_b0']),
                                p['emb_W1'], b2(p['emb_b1']), p['edge_W'])
    gw = (p['Wz'], p['Uz'], b2(p['bz']), p['Wr'], p['Ur'], b2(p['br']),
          p['Wh'], p['Uh'], b2(p['bh']))
    sc_edge = _get_sc_edge()
    for s in range(STEPS):
        agg_lo, agg_hi = sc_edge(alo3.reshape(T * N, HH),
                                 ahi3.reshape(T * N, HH), ia, zer)
        agg_lo = agg_lo[:N]
        agg_hi = agg_hi[:N]
        if s < STEPS - 1:
            h, alo3, ahi3 = _gru_acts_call(h, agg_lo, agg_hi, *gw,
                                           p['edge_W'])
        else:
            h = _gru_last_call(h, agg_lo, agg_hi, *gw)

    ge = _readout_call(h, graph_ids.reshape(N, 1),
                       p['fm_W0'], b2(p['fm_b0']), p['fm_W1'], b2(p['fm_b1']),
                       p['fm_W2'], b2(p['fm_b2']),
                       p['gm_W0'], b2(p['gm_b0']), p['gm_W1'], b2(p['gm_b1']),
                       p['gm_W2'], b2(p['gm_b2']))
    return _final_call(ge, aux_in,
                       p['red_W0'], b2(p['red_b0']), p['red_W1'],
                       b2(p['red_b1']), p['red_W2'], b2(p['red_b2']),
                       p['aux1_W'][:G], p['aux1_W'][G:], b2(p['aux1_b']),
                       p['aux2_W'], b2(p['aux2_b']))


# speedup vs baseline: 2.8445x; 1.9294x over previous
import functools

import jax
import jax.numpy as jnp
from jax import lax
from jax.experimental import pallas as pl
from jax.experimental.pallas import tpu as pltpu
from jax.experimental.pallas import tpu_sc as plsc

N = 100000
E = 1600000
T = 4
H = 32
HH = 16
D_IN = 128
STEPS = 8
G = 64
AUX = 2

NT = 102400
N8 = NT // 8
BN8 = 256
NB8 = N8 // BN8

ROWS = 12800
EPAD = ROWS * 128
RPT = ROWS // 16
CH = 5
CHUNKS = RPT // CH
TPAIRS = CHUNKS // 2
NPAD = NT + 16
TROWS = NPAD // 16

_f32 = jnp.float32


def _sc_edge_body(alo, ahi, ia, zer, olo, ohi,
                  gbufA, dbufA, gbufB, dbufB, rbufA, rbufB,
                  accum, isem, gsem, ssem):
    cid = lax.axis_index("c")

    def half(acts, out):
        sid = lax.axis_index("s")
        pltpu.sync_copy(zer.at[pl.ds(sid * TROWS, TROWS)],
                        accum.at[pl.ds(sid * TROWS, TROWS)])
        plsc.subcore_barrier()
        base = sid * RPT

        CE = CH * 128

        def i_fire(gb, db, c):
            pltpu.async_copy(ia.at[0, pl.ds((base + c * CH) * 128, CE)],
                             gb, isem)
            pltpu.async_copy(ia.at[1, pl.ds((base + c * CH) * 128, CE)],
                             db, isem)

        def i_wait(gb, db, c):
            pltpu.make_async_copy(
                ia.at[0, pl.ds((base + c * CH) * 128, CE)], gb, isem).wait()
            pltpu.make_async_copy(
                ia.at[1, pl.ds((base + c * CH) * 128, CE)], db, isem).wait()

        def g_fire(gb, rb):
            pltpu.async_copy(acts.at[gb], rb, gsem)

        def g_wait(gb, rb):
            pltpu.make_async_copy(acts.at[gb], rb, gsem).wait()

        def s_fire(db, rb):
            pltpu.async_copy(rb, accum.at[db], ssem, add=True)

        def s_wait(db, rb):
            pltpu.make_async_copy(rb, accum.at[db], ssem).wait()

        i_fire(gbufA, dbufA, 0)
        i_wait(gbufA, dbufA, 0)
        g_fire(gbufA, rbufA)

        def body(t, c):
            b = 2 * t + 1

            @pl.when(t > 0)
            def _():
                s_wait(dbufB, rbufB)

            i_fire(gbufB, dbufB, b)
            g_wait(gbufA, rbufA)
            s_fire(dbufA, rbufA)
            i_wait(gbufB, dbufB, b)
            g_fire(gbufB, rbufB)
            s_wait(dbufA, rbufA)

            @pl.when(t < TPAIRS - 1)
            def _():
                i_fire(gbufA, dbufA, b + 1)

            g_wait(gbufB, rbufB)
            s_fire(dbufB, rbufB)

            @pl.when(t < TPAIRS - 1)
            def _():
                i_wait(gbufA, dbufA, b + 1)
                g_fire(gbufA, rbufA)
            return c

        lax.fori_loop(0, TPAIRS, body, 0)
        s_wait(dbufB, rbufB)
        plsc.subcore_barrier()
        pltpu.sync_copy(accum.at[pl.ds(sid * TROWS, TROWS)],
                        out.at[pl.ds(sid * TROWS, TROWS)])

    @pl.when(cid == 0)
    def _():
        half(alo, olo)

    @pl.when(cid == 1)
    def _():
        half(ahi, ohi)


@functools.cache
def _get_sc_edge():
    mesh = plsc.VectorSubcoreMesh(
        core_axis_name="c", subcore_axis_name="s",
        num_cores=2, num_subcores=16)
    return pl.kernel(
        _sc_edge_body,
        out_type=[jax.ShapeDtypeStruct((NPAD, HH), _f32),
                  jax.ShapeDtypeStruct((NPAD, HH), _f32)],
        mesh=mesh,
        scratch_types=[
            pltpu.VMEM((CH * 128,), jnp.int32),
            pltpu.VMEM((CH * 128,), jnp.int32),
            pltpu.VMEM((CH * 128,), jnp.int32),
            pltpu.VMEM((CH * 128,), jnp.int32),
            pltpu.VMEM((CH * 128, HH), _f32),
            pltpu.VMEM((CH * 128, HH), _f32),
            pltpu.VMEM_SHARED((NPAD, HH), _f32),
            pltpu.SemaphoreType.DMA,
            pltpu.SemaphoreType.DMA,
            pltpu.SemaphoreType.DMA,
        ],
        compiler_params=pltpu.CompilerParams(use_tc_tiling_on_sc=False),
    )


def _dot(a, b):
    return jnp.dot(a, b, preferred_element_type=_f32)


def _acts_out(hp, elo_ref, ehi_ref, alo_ref, ahi_ref):
    for t in range(T):
        alo_ref[t] = _dot(hp, elo_ref[t])
        ahi_ref[t] = _dot(hp, ehi_ref[t])


def _embed_body(x_ref, w0, b0, w1, b1, elo_ref, ehi_ref,
                h_ref, alo_ref, ahi_ref):
    hp = jax.nn.relu(_dot(x_ref[...], w0[...]) + b0[...])
    hp = jax.nn.relu(_dot(hp, w1[...]) + b1[...])
    h_ref[...] = hp
    _acts_out(hp, elo_ref, ehi_ref, alo_ref, ahi_ref)


def _full(shape):
    nd = len(shape)
    return pl.BlockSpec(shape, lambda i, _nd=nd: (0,) * _nd)


_embed_call = pl.pallas_call(
    _embed_body,
    grid=(NB8,),
    in_specs=[
        pl.BlockSpec((BN8, 8 * D_IN), lambda i: (i, 0)),
        _full((8 * D_IN, 8 * H)), _full((1, 8 * H)),
        _full((8 * H, 8 * H)), _full((1, 8 * H)),
        _full((T, 8 * H, 8 * HH)), _full((T, 8 * H, 8 * HH)),
    ],
    out_specs=[
        pl.BlockSpec((BN8, 8 * H), lambda i: (i, 0)),
        pl.BlockSpec((T, BN8, 8 * HH), lambda i: (0, i, 0)),
        pl.BlockSpec((T, BN8, 8 * HH), lambda i: (0, i, 0)),
    ],
    out_shape=[
        jax.ShapeDtypeStruct((N8, 8 * H), _f32),
        jax.ShapeDtypeStruct((T, N8, 8 * HH), _f32),
        jax.ShapeDtypeStruct((T, N8, 8 * HH), _f32),
    ],
)


def _gru_math(h_ref, alo_ref, ahi_ref, wzl, wzh, uz, bz,
              wrl, wrh, ur, br, whl, whh, uh, bh):
    hp = h_ref[...]
    al = alo_ref[...]
    ah = ahi_ref[...]

    def am(wl, wh_):
        return _dot(al, wl[...]) + _dot(ah, wh_[...])

    z = jax.nn.sigmoid(am(wzl, wzh) + _dot(hp, uz[...]) + bz[...])
    r = jax.nn.sigmoid(am(wrl, wrh) + _dot(hp, ur[...]) + br[...])
    hh = jnp.tanh(am(whl, whh) + _dot(r * hp, uh[...]) + bh[...])
    return (1.0 - z) * hp + z * hh


def _gru_acts_body(h_ref, alo_ref, ahi_ref, wzl, wzh, uz, bz,
                   wrl, wrh, ur, br, whl, whh, uh, bh,
                   elo_ref, ehi_ref, ho_ref, aol_ref, aoh_ref):
    hn = _gru_math(h_ref, alo_ref, ahi_ref, wzl, wzh, uz, bz,
                   wrl, wrh, ur, br, whl, whh, uh, bh)
    ho_ref[...] = hn
    _acts_out(hn, elo_ref, ehi_ref, aol_ref, aoh_ref)


def _gru_last_body(h_ref, alo_ref, ahi_ref, wzl, wzh, uz, bz,
                   wrl, wrh, ur, br, whl, whh, uh, bh, ho_ref):
    ho_ref[...] = _gru_math(h_ref, alo_ref, ahi_ref, wzl, wzh, uz, bz,
                            wrl, wrh, ur, br, whl, whh, uh, bh)


_gru_in_specs = [
    pl.BlockSpec((BN8, 8 * H), lambda i: (i, 0)),
    pl.BlockSpec((BN8, 8 * HH), lambda i: (i, 0)),
    pl.BlockSpec((BN8, 8 * HH), lambda i: (i, 0)),
] + [_full((8 * HH, 8 * H)), _full((8 * HH, 8 * H)),
     _full((8 * H, 8 * H)), _full((1, 8 * H))] * 3

_gru_acts_call = pl.pallas_call(
    _gru_acts_body,
    grid=(NB8,),
    in_specs=_gru_in_specs + [_full((T, 8 * H, 8 * HH)),
                              _full((T, 8 * H, 8 * HH))],
    out_specs=[
        pl.BlockSpec((BN8, 8 * H), lambda i: (i, 0)),
        pl.BlockSpec((T, BN8, 8 * HH), lambda i: (0, i, 0)),
        pl.BlockSpec((T, BN8, 8 * HH), lambda i: (0, i, 0)),
    ],
    out_shape=[
        jax.ShapeDtypeStruct((N8, 8 * H), _f32),
        jax.ShapeDtypeStruct((T, N8, 8 * HH), _f32),
        jax.ShapeDtypeStruct((T, N8, 8 * HH), _f32),
    ],
)

_gru_last_call = pl.pallas_call(
    _gru_last_body,
    grid=(NB8,),
    in_specs=list(_gru_in_specs),
    out_specs=pl.BlockSpec((BN8, 8 * H), lambda i: (i, 0)),
    out_shape=jax.ShapeDtypeStruct((N8, 8 * H), _f32),
)


def _mlp3(v, w0, b0, w1, b1, w2, b2):
    v = jax.nn.relu(_dot(v, w0[...]) + b0[...])
    v = jax.nn.relu(_dot(v, w1[...]) + b1[...])
    return _dot(v, w2[...]) + b2[...]


def _readout_body(h_ref, gid_ref, fw0, fb0, fw1, fb1, fw2, fb2,
                  gw0, gb0, gw1, gb1, gw2, gb2, ge_ref):
    hp = h_ref[...]
    f = _mlp3(hp, fw0, fb0, fw1, fb1, fw2, fb2)
    g = jax.nn.sigmoid(_mlp3(hp, gw0, gb0, gw1, gb1, gw2, gb2))
    gated = g * f
    ids = gid_ref[...]

    part = jnp.zeros((G, G), _f32)
    for s in range(8):
        oh = (ids[:, s:s + 1]
              == lax.broadcasted_iota(jnp.int32, (BN8, G), 1)).astype(_f32)
        part += lax.dot_general(oh, gated[:, s * G:(s + 1) * G],
                                (((0,), (0,)), ((), ())),
                                preferred_element_type=_f32)

    @pl.when(pl.program_id(0) == 0)
    def _():
        ge_ref[...] = jnp.zeros_like(ge_ref)

    ge_ref[...] += part


_readout_call = pl.pallas_call(
    _readout_body,
    grid=(NB8,),
    in_specs=[
        pl.BlockSpec((BN8, 8 * H), lambda i: (i, 0)),
        pl.BlockSpec((BN8, 8), lambda i: (i, 0)),
        _full((8 * H, 8 * H)), _full((1, 8 * H)),
        _full((8 * H, 8 * H)), _full((1, 8 * H)),
        _full((8 * H, 8 * G)), _full((1, 8 * G)),
        _full((8 * H, 8 * H)), _full((1, 8 * H)),
        _full((8 * H, 8 * H)), _full((1, 8 * H)),
        _full((8 * H, 8 * G)), _full((1, 8 * G)),
    ],
    out_specs=pl.BlockSpec((G, G), lambda i: (0, 0)),
    out_shape=jax.ShapeDtypeStruct((G, G), _f32),
)


def _final_body(ge_ref, aux_ref, rw0, rb0, rw1, rb1, rw2, rb2,
                a1wa, a1wb, a1b, a2w, a2b, out_ref):
    r1 = _mlp3(ge_ref[...], rw0, rb0, rw1, rb1, rw2, rb2)
    a1 = jax.nn.relu(_dot(r1, a1wa[...]) + _dot(aux_ref[...], a1wb[...])
                     + a1b[...])
    out_ref[...] = jax.nn.sigmoid(_dot(a1, a2w[...]) + a2b[...])


_final_call = pl.pallas_call(
    _final_body,
    out_shape=jax.ShapeDtypeStruct((G, AUX), _f32),
)


def kernel(x, edge_index, edge_type, graph_ids, aux_in, params):
    p = params
    eye8 = jnp.eye(8, dtype=_f32)

    def bd(w):
        return jnp.kron(eye8, w)

    def b8(v):
        return jnp.tile(v, 8).reshape(1, -1)

    src = edge_index[0]
    dst = edge_index[1]
    gidx = edge_type * NT + src
    padn = EPAD - E
    gidx2 = jnp.concatenate([gidx, jnp.zeros((padn,), jnp.int32)])
    dst2 = jnp.concatenate([dst, jnp.full((padn,), NT, jnp.int32)])
    ia = jnp.stack([gidx2, dst2], axis=0)
    zer = jnp.zeros((NPAD, HH), _f32)

    xp = jnp.concatenate(
        [x, jnp.zeros((NT - N, D_IN), _f32)]).reshape(N8, 8 * D_IN)
    gid8 = jnp.concatenate(
        [graph_ids, jnp.full((NT - N,), G, jnp.int32)]).reshape(N8, 8)

    ew = p['edge_W']
    elo = jnp.stack([bd(ew[t][:, :HH]) for t in range(T)])
    ehi = jnp.stack([bd(ew[t][:, HH:]) for t in range(T)])

    hp, alo3, ahi3 = _embed_call(xp, bd(p['emb_W0']), b8(p['emb_b0']),
                                 bd(p['emb_W1']), b8(p['emb_b1']), elo, ehi)
    gw = (bd(p['Wz'][:HH]), bd(p['Wz'][HH:]), bd(p['Uz']), b8(p['bz']),
          bd(p['Wr'][:HH]), bd(p['Wr'][HH:]), bd(p['Ur']), b8(p['br']),
          bd(p['Wh'][:HH]), bd(p['Wh'][HH:]), bd(p['Uh']), b8(p['bh']))
    sc_edge = _get_sc_edge()
    for s in range(STEPS):
        agg_lo, agg_hi = sc_edge(alo3.reshape(T * NT, HH),
                                 ahi3.reshape(T * NT, HH), ia, zer)
        aglo = agg_lo.reshape(NPAD // 8, 8 * HH)
        aghi = agg_hi.reshape(NPAD // 8, 8 * HH)
        if s < STEPS - 1:
            hp, alo3, ahi3 = _gru_acts_call(hp, aglo, aghi, *gw, elo, ehi)
        else:
            hp = _gru_last_call(hp, aglo, aghi, *gw)

    ge = _readout_call(hp, gid8,
                       bd(p['fm_W0']), b8(p['fm_b0']),
                       bd(p['fm_W1']), b8(p['fm_b1']),
                       bd(p['fm_W2']), b8(p['fm_b2']),
                       bd(p['gm_W0']), b8(p['gm_b0']),
                       bd(p['gm_W1']), b8(p['gm_b1']),
                       bd(p['gm_W2']), b8(p['gm_b2']))
    return _final_call(ge, aux_in,
                       p['red_W0'], p['red_b0'].reshape(1, -1),
                       p['red_W1'], p['red_b1'].reshape(1, -1),
                       p['red_W2'], p['red_b2'].reshape(1, -1),
                       p['aux1_W'][:G], p['aux1_W'][G:],
                       p['aux1_b'].reshape(1, -1),
                       p['aux2_W'], p['aux2_b'].reshape(1, -1))


# speedup vs baseline: 2.8661x; 1.0076x over previous
import functools

import jax
import jax.numpy as jnp
from jax import lax
from jax.experimental import pallas as pl
from jax.experimental.pallas import tpu as pltpu
from jax.experimental.pallas import tpu_sc as plsc

N = 100000
E = 1600000
T = 4
H = 32
HH = 16
D_IN = 128
STEPS = 8
G = 64
AUX = 2

NT = 102400
N8 = NT // 8
BN8 = 256
NB8 = N8 // BN8

ROWS = 12800
EPAD = ROWS * 128
RPT = ROWS // 16
CH = 5
CHUNKS = RPT // CH
TPAIRS = CHUNKS // 2
NPAD = NT + 16
TROWS = NPAD // 16

_f32 = jnp.float32


def _sc_edge_body(alo, ahi, gl, dl, zer, olo, ohi,
                  gbufA, dbufA, gbufB, dbufB, rbufA, rbufB,
                  accum, isem, gsem, ssem):
    cid = lax.axis_index("c")

    def half(acts, out):
        sid = lax.axis_index("s")
        pltpu.sync_copy(zer.at[pl.ds(sid * TROWS, TROWS)],
                        accum.at[pl.ds(sid * TROWS, TROWS)])
        plsc.subcore_barrier()
        base = sid * RPT

        CE = CH * 128

        def i_fire(gb, db, c):
            pltpu.async_copy(gl.at[pl.ds((base + c * CH) * 128, CE)],
                             gb, isem)
            pltpu.async_copy(dl.at[pl.ds((base + c * CH) * 128, CE)],
                             db, isem)

        def i_wait(gb, db, c):
            pltpu.make_async_copy(
                gl.at[pl.ds((base + c * CH) * 128, CE)], gb, isem).wait()
            pltpu.make_async_copy(
                dl.at[pl.ds((base + c * CH) * 128, CE)], db, isem).wait()

        def g_fire(gb, rb):
            pltpu.async_copy(acts.at[gb], rb, gsem)

        def g_wait(gb, rb):
            pltpu.make_async_copy(acts.at[gb], rb, gsem).wait()

        def s_fire(db, rb):
            pltpu.async_copy(rb, accum.at[db], ssem, add=True)

        def s_wait(db, rb):
            pltpu.make_async_copy(rb, accum.at[db], ssem).wait()

        i_fire(gbufA, dbufA, 0)
        i_wait(gbufA, dbufA, 0)
        g_fire(gbufA, rbufA)

        def body(t, c):
            b = 2 * t + 1

            @pl.when(t > 0)
            def _():
                s_wait(dbufB, rbufB)

            i_fire(gbufB, dbufB, b)
            g_wait(gbufA, rbufA)
            s_fire(dbufA, rbufA)
            i_wait(gbufB, dbufB, b)
            g_fire(gbufB, rbufB)
            s_wait(dbufA, rbufA)

            @pl.when(t < TPAIRS - 1)
            def _():
                i_fire(gbufA, dbufA, b + 1)

            g_wait(gbufB, rbufB)
            s_fire(dbufB, rbufB)

            @pl.when(t < TPAIRS - 1)
            def _():
                i_wait(gbufA, dbufA, b + 1)
                g_fire(gbufA, rbufA)
            return c

        lax.fori_loop(0, TPAIRS, body, 0)
        s_wait(dbufB, rbufB)
        plsc.subcore_barrier()
        pltpu.sync_copy(accum.at[pl.ds(sid * TROWS, TROWS)],
                        out.at[pl.ds(sid * TROWS, TROWS)])

    @pl.when(cid == 0)
    def _():
        half(alo, olo)

    @pl.when(cid == 1)
    def _():
        half(ahi, ohi)


@functools.cache
def _get_sc_edge():
    mesh = plsc.VectorSubcoreMesh(
        core_axis_name="c", subcore_axis_name="s",
        num_cores=2, num_subcores=16)
    return pl.kernel(
        _sc_edge_body,
        out_type=[jax.ShapeDtypeStruct((NPAD, HH), _f32),
                  jax.ShapeDtypeStruct((NPAD, HH), _f32)],
        mesh=mesh,
        scratch_types=[
            pltpu.VMEM((CH * 128,), jnp.int32),
            pltpu.VMEM((CH * 128,), jnp.int32),
            pltpu.VMEM((CH * 128,), jnp.int32),
            pltpu.VMEM((CH * 128,), jnp.int32),
            pltpu.VMEM((CH * 128, HH), _f32),
            pltpu.VMEM((CH * 128, HH), _f32),
            pltpu.VMEM_SHARED((NPAD, HH), _f32),
            pltpu.SemaphoreType.DMA,
            pltpu.SemaphoreType.DMA,
            pltpu.SemaphoreType.DMA,
        ],
        compiler_params=pltpu.CompilerParams(use_tc_tiling_on_sc=False),
    )


def _dot(a, b):
    return jnp.dot(a, b, preferred_element_type=_f32)


def _acts_out(hp, elo_ref, ehi_ref, alo_ref, ahi_ref):
    for t in range(T):
        alo_ref[t] = _dot(hp, elo_ref[t])
        ahi_ref[t] = _dot(hp, ehi_ref[t])


def _embed_body(x_ref, w0, b0, w1, b1, elo_ref, ehi_ref,
                h_ref, alo_ref, ahi_ref):
    hp = jax.nn.relu(_dot(x_ref[...], w0[...]) + b0[...])
    hp = jax.nn.relu(_dot(hp, w1[...]) + b1[...])
    h_ref[...] = hp
    _acts_out(hp, elo_ref, ehi_ref, alo_ref, ahi_ref)


def _full(shape):
    nd = len(shape)
    return pl.BlockSpec(shape, lambda i, _nd=nd: (0,) * _nd)


_embed_call = pl.pallas_call(
    _embed_body,
    grid=(NB8,),
    in_specs=[
        pl.BlockSpec((BN8, 8 * D_IN), lambda i: (i, 0)),
        _full((8 * D_IN, 8 * H)), _full((1, 8 * H)),
        _full((8 * H, 8 * H)), _full((1, 8 * H)),
        _full((T, 8 * H, 8 * HH)), _full((T, 8 * H, 8 * HH)),
    ],
    out_specs=[
        pl.BlockSpec((BN8, 8 * H), lambda i: (i, 0)),
        pl.BlockSpec((T, BN8, 8 * HH), lambda i: (0, i, 0)),
        pl.BlockSpec((T, BN8, 8 * HH), lambda i: (0, i, 0)),
    ],
    out_shape=[
        jax.ShapeDtypeStruct((N8, 8 * H), _f32),
        jax.ShapeDtypeStruct((T, N8, 8 * HH), _f32),
        jax.ShapeDtypeStruct((T, N8, 8 * HH), _f32),
    ],
)


def _gru_math(h_ref, alo_ref, ahi_ref, wzl, wzh, uz, bz,
              wrl, wrh, ur, br, whl, whh, uh, bh):
    hp = h_ref[...]
    al = alo_ref[...]
    ah = ahi_ref[...]

    def am(wl, wh_):
        return _dot(al, wl[...]) + _dot(ah, wh_[...])

    z = jax.nn.sigmoid(am(wzl, wzh) + _dot(hp, uz[...]) + bz[...])
    r = jax.nn.sigmoid(am(wrl, wrh) + _dot(hp, ur[...]) + br[...])
    hh = jnp.tanh(am(whl, whh) + _dot(r * hp, uh[...]) + bh[...])
    return (1.0 - z) * hp + z * hh


def _gru_acts_body(h_ref, alo_ref, ahi_ref, wzl, wzh, uz, bz,
                   wrl, wrh, ur, br, whl, whh, uh, bh,
                   elo_ref, ehi_ref, ho_ref, aol_ref, aoh_ref):
    hn = _gru_math(h_ref, alo_ref, ahi_ref, wzl, wzh, uz, bz,
                   wrl, wrh, ur, br, whl, whh, uh, bh)
    ho_ref[...] = hn
    _acts_out(hn, elo_ref, ehi_ref, aol_ref, aoh_ref)


def _gru_last_body(h_ref, alo_ref, ahi_ref, wzl, wzh, uz, bz,
                   wrl, wrh, ur, br, whl, whh, uh, bh, ho_ref):
    ho_ref[...] = _gru_math(h_ref, alo_ref, ahi_ref, wzl, wzh, uz, bz,
                            wrl, wrh, ur, br, whl, whh, uh, bh)


_gru_in_specs = [
    pl.BlockSpec((BN8, 8 * H), lambda i: (i, 0)),
    pl.BlockSpec((BN8, 8 * HH), lambda i: (i, 0)),
    pl.BlockSpec((BN8, 8 * HH), lambda i: (i, 0)),
] + [_full((8 * HH, 8 * H)), _full((8 * HH, 8 * H)),
     _full((8 * H, 8 * H)), _full((1, 8 * H))] * 3

_gru_acts_call = pl.pallas_call(
    _gru_acts_body,
    grid=(NB8,),
    in_specs=_gru_in_specs + [_full((T, 8 * H, 8 * HH)),
                              _full((T, 8 * H, 8 * HH))],
    out_specs=[
        pl.BlockSpec((BN8, 8 * H), lambda i: (i, 0)),
        pl.BlockSpec((T, BN8, 8 * HH), lambda i: (0, i, 0)),
        pl.BlockSpec((T, BN8, 8 * HH), lambda i: (0, i, 0)),
    ],
    out_shape=[
        jax.ShapeDtypeStruct((N8, 8 * H), _f32),
        jax.ShapeDtypeStruct((T, N8, 8 * HH), _f32),
        jax.ShapeDtypeStruct((T, N8, 8 * HH), _f32),
    ],
)

_gru_last_call = pl.pallas_call(
    _gru_last_body,
    grid=(NB8,),
    in_specs=list(_gru_in_specs),
    out_specs=pl.BlockSpec((BN8, 8 * H), lambda i: (i, 0)),
    out_shape=jax.ShapeDtypeStruct((N8, 8 * H), _f32),
)


def _mlp3(v, w0, b0, w1, b1, w2, b2):
    v = jax.nn.relu(_dot(v, w0[...]) + b0[...])
    v = jax.nn.relu(_dot(v, w1[...]) + b1[...])
    return _dot(v, w2[...]) + b2[...]


def _readout_body(h_ref, gid_ref, fw0, fb0, fw1, fb1, fw2, fb2,
                  gw0, gb0, gw1, gb1, gw2, gb2, ge_ref):
    hp = h_ref[...]
    f = _mlp3(hp, fw0, fb0, fw1, fb1, fw2, fb2)
    g = jax.nn.sigmoid(_mlp3(hp, gw0, gb0, gw1, gb1, gw2, gb2))
    gated = g * f
    ids = gid_ref[...]

    part = jnp.zeros((G, G), _f32)
    for s in range(8):
        oh = (ids[:, s:s + 1]
              == lax.broadcasted_iota(jnp.int32, (BN8, G), 1)).astype(_f32)
        part += lax.dot_general(oh, gated[:, s * G:(s + 1) * G],
                                (((0,), (0,)), ((), ())),
                                preferred_element_type=_f32)

    @pl.when(pl.program_id(0) == 0)
    def _():
        ge_ref[...] = jnp.zeros_like(ge_ref)

    ge_ref[...] += part


_readout_call = pl.pallas_call(
    _readout_body,
    grid=(NB8,),
    in_specs=[
        pl.BlockSpec((BN8, 8 * H), lambda i: (i, 0)),
        pl.BlockSpec((BN8, 8), lambda i: (i, 0)),
        _full((8 * H, 8 * H)), _full((1, 8 * H)),
        _full((8 * H, 8 * H)), _full((1, 8 * H)),
        _full((8 * H, 8 * G)), _full((1, 8 * G)),
        _full((8 * H, 8 * H)), _full((1, 8 * H)),
        _full((8 * H, 8 * H)), _full((1, 8 * H)),
        _full((8 * H, 8 * G)), _full((1, 8 * G)),
    ],
    out_specs=pl.BlockSpec((G, G), lambda i: (0, 0)),
    out_shape=jax.ShapeDtypeStruct((G, G), _f32),
)


def _final_body(ge_ref, aux_ref, rw0, rb0, rw1, rb1, rw2, rb2,
                a1wa, a1wb, a1b, a2w, a2b, out_ref):
    r1 = _mlp3(ge_ref[...], rw0, rb0, rw1, rb1, rw2, rb2)
    a1 = jax.nn.relu(_dot(r1, a1wa[...]) + _dot(aux_ref[...], a1wb[...])
                     + a1b[...])
    out_ref[...] = jax.nn.sigmoid(_dot(a1, a2w[...]) + a2b[...])


_final_call = pl.pallas_call(
    _final_body,
    out_shape=jax.ShapeDtypeStruct((G, AUX), _f32),
)


def kernel(x, edge_index, edge_type, graph_ids, aux_in, params):
    p = params
    eye8 = jnp.eye(8, dtype=_f32)

    def bd(w):
        return jnp.kron(eye8, w)

    def b8(v):
        return jnp.tile(v, 8).reshape(1, -1)

    src = edge_index[0]
    dst = edge_index[1]
    gidx = edge_type * NT + src
    padn = EPAD - E
    gidx2 = jnp.concatenate([gidx, jnp.zeros((padn,), jnp.int32)])
    dst2 = jnp.concatenate([dst, jnp.full((padn,), NT, jnp.int32)])
    zer = jnp.zeros((NPAD, HH), _f32)

    xp = jnp.concatenate(
        [x, jnp.zeros((NT - N, D_IN), _f32)]).reshape(N8, 8 * D_IN)
    gid8 = jnp.concatenate(
        [graph_ids, jnp.full((NT - N,), G, jnp.int32)]).reshape(N8, 8)

    ew = p['edge_W']
    elo = jnp.stack([bd(ew[t][:, :HH]) for t in range(T)])
    ehi = jnp.stack([bd(ew[t][:, HH:]) for t in range(T)])

    hp, alo3, ahi3 = _embed_call(xp, bd(p['emb_W0']), b8(p['emb_b0']),
                                 bd(p['emb_W1']), b8(p['emb_b1']), elo, ehi)
    gw = (bd(p['Wz'][:HH]), bd(p['Wz'][HH:]), bd(p['Uz']), b8(p['bz']),
          bd(p['Wr'][:HH]), bd(p['Wr'][HH:]), bd(p['Ur']), b8(p['br']),
          bd(p['Wh'][:HH]), bd(p['Wh'][HH:]), bd(p['Uh']), b8(p['bh']))
    sc_edge = _get_sc_edge()
    for s in range(STEPS):
        agg_lo, agg_hi = sc_edge(alo3.reshape(T * NT, HH),
                                 ahi3.reshape(T * NT, HH), gidx2, dst2, zer)
        aglo = agg_lo.reshape(NPAD // 8, 8 * HH)
        aghi = agg_hi.reshape(NPAD // 8, 8 * HH)
        if s < STEPS - 1:
            hp, alo3, ahi3 = _gru_acts_call(hp, aglo, aghi, *gw, elo, ehi)
        else:
            hp = _gru_last_call(hp, aglo, aghi, *gw)

    ge = _readout_call(hp, gid8,
                       bd(p['fm_W0']), b8(p['fm_b0']),
                       bd(p['fm_W1']), b8(p['fm_b1']),
                       bd(p['fm_W2']), b8(p['fm_b2']),
                       bd(p['gm_W0']), b8(p['gm_b0']),
                       bd(p['gm_W1']), b8(p['gm_b1']),
                       bd(p['gm_W2']), b8(p['gm_b2']))
    return _final_call(ge, aux_in,
                       p['red_W0'], p['red_b0'].reshape(1, -1),
                       p['red_W1'], p['red_b1'].reshape(1, -1),
                       p['red_W2'], p['red_b2'].reshape(1, -1),
                       p['aux1_W'][:G], p['aux1_W'][G:],
                       p['aux1_b'].reshape(1, -1),
                       p['aux2_W'], p['aux2_b'].reshape(1, -1))
